# baseline (speedup 1.0000x reference)
import jax
import jax.numpy as jnp
from jax import lax
from jax.experimental import pallas as pl
from jax.experimental.pallas import tpu as pltpu

_NEG_POS_RATIO = 3
_SENTINEL = 0x7F800000
_B, _N = 32, 20000
_G = 4
_RB = _B // _G


def _body(cp_ref, ct_ref, lp_ref, lt_ref, ctr_ref, out_ref, keys_vmem, acc):
    j = pl.program_id(0)

    @pl.when(j == 0)
    def _():
        acc[0] = 0.0
        acc[1] = 0.0
        acc[2] = 0.0
        acc[3] = 0.0

    ct = ct_ref[...]
    pos = ct > 0.5
    posf = pos.astype(jnp.float32)
    x = cp_ref[...]
    cls_elem = jnp.maximum(x, 0.0) - x * ct + jnp.log1p(jnp.exp(-jnp.abs(x)))
    keys = lax.bitcast_convert_type(cls_elem, jnp.int32)
    keys_vmem[pl.ds(j * _RB, _RB), :] = jnp.where(pos, jnp.int32(_SENTINEL), keys)

    posfr = (ctr_ref[...] > 0.5).astype(jnp.float32)
    d = lp_ref[...] - lt_ref[...]
    ad = jnp.abs(d)
    sl = jnp.where(ad < 1.0, 0.5 * d * d, ad - 0.5)

    acc[0] += jnp.sum(posf)
    acc[1] += jnp.sum(posfr * sl)
    acc[2] += jnp.sum(posf * cls_elem)
    acc[3] += jnp.sum(cls_elem)

    @pl.when(j == _G - 1)
    def _():
        num_pos_f = acc[0]
        p_i = num_pos_f.astype(jnp.int32)
        k = jnp.maximum(jnp.minimum(_NEG_POS_RATIO * p_i, _N - p_i), 0)

        def do_select():
            keys_all = keys_vmem[...]

            def bit_step(t, carry):
                prefix, kk = carry
                bit = 30 - t
                m = (keys_all >> bit) == (prefix >> bit)
                c0 = jnp.sum(m.astype(jnp.int32), axis=1, keepdims=True)
                take0 = kk <= c0
                prefix = jnp.where(take0, prefix,
                                   prefix | (jnp.int32(1) << bit))
                kk = jnp.where(take0, kk, kk - c0)
                return prefix, kk

            thresh, _ = lax.fori_loop(
                0, 31, bit_step,
                (jnp.zeros((_B, 1), jnp.int32), jnp.full((_B, 1), k)),
            )

            less = keys_all < thresh
            cnt_less = jnp.sum(less.astype(jnp.int32), axis=1, keepdims=True)
            vals = lax.bitcast_convert_type(keys_all, jnp.float32)
            sum_less = jnp.sum(jnp.where(less, vals, 0.0), axis=1,
                               keepdims=True)
            v_t = lax.bitcast_convert_type(thresh, jnp.float32)
            sel = sum_less + (k.astype(jnp.float32)
                              - cnt_less.astype(jnp.float32)) * v_t
            return jnp.sum(sel)

        sel_total = lax.cond(k > 0, do_select, lambda: jnp.float32(0.0))

        num_pos_safe = jnp.maximum(num_pos_f, 1.0)
        total = (acc[1] + acc[2] + sel_total) / num_pos_safe
        zero_branch = acc[3] / jnp.float32(_B * _N)
        result = jnp.where(num_pos_f == 0.0, zero_branch, total)
        out_ref[...] = jnp.broadcast_to(result, (1, 1))


def kernel(loc_preds, loc_targets, cls_preds, cls_targets):
    lp2 = loc_preds.reshape(_B, 4 * _N)
    lt2 = loc_targets.reshape(_B, 4 * _N)
    ctr = jnp.repeat(cls_targets, 4, axis=1)
    out = pl.pallas_call(
        _body,
        grid=(_G,),
        in_specs=[
            pl.BlockSpec((_RB, _N), lambda j: (j, 0)),
            pl.BlockSpec((_RB, _N), lambda j: (j, 0)),
            pl.BlockSpec((_RB, 4 * _N), lambda j: (j, 0)),
            pl.BlockSpec((_RB, 4 * _N), lambda j: (j, 0)),
            pl.BlockSpec((_RB, 4 * _N), lambda j: (j, 0)),
        ],
        out_specs=pl.BlockSpec((1, 1), lambda j: (0, 0)),
        out_shape=jax.ShapeDtypeStruct((1, 1), jnp.float32),
        scratch_shapes=[
            pltpu.VMEM((_B, _N), jnp.int32),
            pltpu.SMEM((4,), jnp.float32),
        ],
    )(cls_preds, cls_targets, lp2, lt2, ctr)
    return out[0, 0]


# speedup vs baseline: 4.5329x; 4.5329x over previous
import jax
import jax.numpy as jnp
from jax import lax
from jax.experimental import pallas as pl
from jax.experimental.pallas import tpu as pltpu

_NEG_POS_RATIO = 3
_SENTINEL = 0x7F800000


def _ssd_body(lpt_ref, ltt_ref, cp_ref, ct_ref, out_ref):
    B, N = ct_ref.shape
    ct = ct_ref[...]
    pos = ct > 0.5
    posf = pos.astype(jnp.float32)
    num_pos_i = jnp.sum(pos.astype(jnp.int32))

    loc_sum = jnp.float32(0.0)
    for c in range(4):
        d = lpt_ref[c] - ltt_ref[c]
        ad = jnp.abs(d)
        sl = jnp.where(ad < 1.0, 0.5 * d * d, ad - 0.5)
        loc_sum = loc_sum + jnp.sum(posf * sl)

    x = cp_ref[...]
    cls_elem = jnp.maximum(x, 0.0) - x * ct + jnp.log1p(jnp.exp(-jnp.abs(x)))
    cls_all_sum = jnp.sum(cls_elem)
    cls_pos_sum = jnp.sum(posf * cls_elem)

    k = jnp.maximum(jnp.minimum(_NEG_POS_RATIO * num_pos_i, N - num_pos_i), 0)

    def do_select():
        keys = lax.bitcast_convert_type(cls_elem, jnp.int32)
        keys = jnp.where(pos, jnp.int32(_SENTINEL), keys)

        def bit_step(i, carry):
            prefix, kk = carry
            bit = 30 - i
            m = (keys >> bit) == (prefix >> bit)
            c0 = jnp.sum(m.astype(jnp.int32), axis=1, keepdims=True)
            take0 = kk <= c0
            prefix = jnp.where(take0, prefix, prefix | (jnp.int32(1) << bit))
            kk = jnp.where(take0, kk, kk - c0)
            return prefix, kk

        thresh, _ = lax.fori_loop(
            0, 31, bit_step,
            (jnp.zeros((B, 1), jnp.int32), jnp.full((B, 1), k, jnp.int32)),
        )

        vals = lax.bitcast_convert_type(keys, jnp.float32)
        v_t = lax.bitcast_convert_type(thresh, jnp.float32)
        less = keys < thresh
        cnt_less = jnp.sum(less.astype(jnp.int32), axis=1, keepdims=True)
        sum_less = jnp.sum(jnp.where(less, vals, 0.0), axis=1, keepdims=True)
        kf = k.astype(jnp.float32)
        sel = sum_less + (kf - cnt_less.astype(jnp.float32)) * v_t
        return jnp.sum(sel)

    select_total = lax.cond(k > 0, do_select, lambda: jnp.float32(0.0))

    num_pos_f = num_pos_i.astype(jnp.float32)
    num_pos_safe = jnp.maximum(num_pos_f, 1.0)
    total = (loc_sum + cls_pos_sum + select_total) / num_pos_safe
    zero_branch = cls_all_sum / jnp.float32(B * N)
    result = jnp.where(num_pos_i == 0, zero_branch, total)
    out_ref[...] = jnp.broadcast_to(result, (1, 1))


def kernel(loc_preds, loc_targets, cls_preds, cls_targets):
    lpt = jnp.transpose(loc_preds, (2, 0, 1))
    ltt = jnp.transpose(loc_targets, (2, 0, 1))
    out = pl.pallas_call(
        _ssd_body,
        out_shape=jax.ShapeDtypeStruct((1, 1), jnp.float32),
    )(lpt, ltt, cls_preds, cls_targets)
    return out[0, 0]
